# Initial kernel scaffold; baseline (speedup 1.0000x reference)
import jax
import jax.numpy as jnp
from jax.experimental import pallas as pl


def kernel(upper_tri_vector):
    raise NotImplementedError("write your pallas kernel here")


# speedup vs baseline: 127.6333x; 127.6333x over previous
import functools

import jax
import jax.numpy as jnp
from jax import lax
from jax.experimental import pallas as pl
from jax.experimental.pallas import tpu as pltpu
from jax.experimental.pallas import tpu_sc as plsc

N = 4096
T = 128
NT = N // T
NTILES = NT * (NT + 1) // 2
BAND = NT + 1
NC = 2
NS = 16
NW = NC * NS
ITERS = (NTILES + NW - 1) // NW
STRIPW = T + 16
PAD = 256


def _body(v_hbm, out_hbm, strip, tile, tile_t, sem):
    wid = lax.axis_index("s") * NC + lax.axis_index("c")
    iota16 = lax.iota(jnp.int32, 16)

    def tile_step(m, carry):
        t = m * NW + wid

        @pl.when(t < NTILES)
        def _():
            b = t // BAND
            p = t - b * BAND
            first = p < (NT - b)
            tile_i = jnp.where(first, b, NT - 1 - b)
            tile_j = jnp.where(first, b + p, p - 1)
            i0 = tile_i * T
            j0 = tile_j * T

            def fire(k, c):
                i = i0 + k
                s = i * N - (i * (i - 1)) // 2 - i + j0
                a = pl.multiple_of((s >> 4) << 4, 16)
                pltpu.async_copy(
                    v_hbm.at[pl.ds(a, STRIPW)],
                    strip.at[pl.ds(k * STRIPW, STRIPW)],
                    sem,
                )
                return c

            lax.fori_loop(0, T, fire, 0)
            pltpu.make_async_copy(
                v_hbm.at[pl.ds(0, T * STRIPW)], strip, sem
            ).wait()

            def rowfix(k, c):
                i = i0 + k
                s = i * N - (i * (i - 1)) // 2 - i + j0
                r = s & 15
                krow = jnp.full((16,), k, jnp.int32)
                base = k * STRIPW + r
                for g in range(T // 16):
                    cols = g * 16 + iota16
                    vals = plsc.load_gather(strip, [base + cols])
                    tile[k, pl.ds(g * 16, 16)] = vals
                    plsc.store_scatter(tile_t, [cols, krow], vals)
                return c

            lax.fori_loop(0, T, rowfix, 0)

            diag = tile_i == tile_j

            @pl.when(diag)
            def _():
                def merge(k, c):
                    for g in range(T // 16):
                        cols = g * 16 + iota16
                        a = tile[k, pl.ds(g * 16, 16)]
                        bt = tile_t[k, pl.ds(g * 16, 16)]
                        tile[k, pl.ds(g * 16, 16)] = jnp.where(cols >= k, a, bt)
                    return c

                lax.fori_loop(0, T, merge, 0)

            i0a = pl.multiple_of(i0, T)
            j0a = pl.multiple_of(j0, T)
            pltpu.sync_copy(tile, out_hbm.at[pl.ds(i0a, T), pl.ds(j0a, T)])

            @pl.when(jnp.logical_not(diag))
            def _():
                pltpu.sync_copy(
                    tile_t, out_hbm.at[pl.ds(j0a, T), pl.ds(i0a, T)]
                )

        return carry

    lax.fori_loop(0, ITERS, tile_step, 0)


@jax.jit
def kernel(upper_tri_vector):
    v_pad = jnp.concatenate(
        [upper_tri_vector, jnp.zeros((PAD,), upper_tri_vector.dtype)]
    )
    mesh = plsc.VectorSubcoreMesh(
        core_axis_name="c", subcore_axis_name="s", num_cores=NC
    )
    fn = pl.kernel(
        _body,
        out_type=jax.ShapeDtypeStruct((N, N), jnp.float32),
        mesh=mesh,
        scratch_types=[
            pltpu.VMEM((T * STRIPW,), jnp.float32),
            pltpu.VMEM((T, T), jnp.float32),
            pltpu.VMEM((T, T), jnp.float32),
            pltpu.SemaphoreType.DMA,
        ],
        compiler_params=pltpu.CompilerParams(needs_layout_passes=False),
    )
    return fn(v_pad)


# speedup vs baseline: 175.4612x; 1.3747x over previous
import jax
import jax.numpy as jnp
from jax import lax
from jax.experimental import pallas as pl
from jax.experimental.pallas import tpu as pltpu
from jax.experimental.pallas import tpu_sc as plsc

N = 4096
T = 128
NT = N // T
NC = 2
NS = 16
NW = NC * NS
STEPS = 17
STRIPW = T + 16
PAD = 256


def _body(v_hbm, out_hbm, strip0, strip1, tile0, tile1, tt0, tt1,
          sem_in0, sem_in1, sem_out0, sem_out1):
    wid = lax.axis_index("s") * NC + lax.axis_index("c")
    iota16 = lax.iota(jnp.int32, 16)
    strips = [strip0, strip1]
    tiles = [tile0, tile1]
    tts = [tt0, tt1]
    sem_ins = [sem_in0, sem_in1]
    sem_outs = [sem_out0, sem_out1]

    def decode(mm):
        first = wid < (NT - mm)
        ti = jnp.where(first, wid, wid - NT + mm)
        tj = jnp.where(first, wid + mm, wid)
        valid = jnp.logical_or(mm < 16, wid < 16)
        return ti, tj, valid

    def row_start(i, j0):
        return i * N - (i * (i - 1)) // 2 - i + j0

    def fire(mm, par):
        ti, tj, _ = decode(mm)
        i0 = ti * T
        j0 = tj * T
        strip = strips[par]
        sem = sem_ins[par]

        def go(k, s):
            a = pl.multiple_of((s >> 4) << 4, 16)
            pltpu.async_copy(
                v_hbm.at[pl.ds(a, STRIPW)],
                strip.at[pl.ds(k * STRIPW, STRIPW)],
                sem,
            )
            return s + (N - 1 - (i0 + k))

        lax.fori_loop(0, T, go, row_start(i0, j0))

    def wait_tilebytes(par, buf):
        pltpu.make_async_copy(
            out_hbm.at[pl.ds(0, T), pl.ds(0, T)], buf, sem_outs[par]
        ).wait()

    def step(m, par):
        pi, pj, pvalid = decode(m - 2)

        @pl.when(jnp.logical_and(m >= 2, pvalid))
        def _():
            wait_tilebytes(par, tiles[par])

            @pl.when(pi != pj)
            def _():
                wait_tilebytes(par, tts[par])

        ni, nj, nvalid = decode(m + 1)

        @pl.when(jnp.logical_and(m + 1 < STEPS, nvalid))
        def _():
            fire(m + 1, 1 - par)

        ti, tj, valid = decode(m)

        @pl.when(valid)
        def _():
            i0 = ti * T
            j0 = tj * T
            strip = strips[par]
            tile = tiles[par]
            tile_t = tts[par]
            pltpu.make_async_copy(
                v_hbm.at[pl.ds(0, T * STRIPW)], strip, sem_ins[par]
            ).wait()

            def rowfix(k, s):
                r = s & 15
                krow = jnp.full((16,), k, jnp.int32)
                base = k * STRIPW + r
                for g in range(T // 16):
                    cols = g * 16 + iota16
                    vals = plsc.load_gather(strip, [base + cols])
                    tile[k, pl.ds(g * 16, 16)] = vals
                    plsc.store_scatter(tile_t, [cols, krow], vals)
                return s + (N - 1 - (i0 + k))

            lax.fori_loop(0, T, rowfix, row_start(i0, j0))

            diag = ti == tj

            @pl.when(diag)
            def _():
                def merge(k, c):
                    for g in range(T // 16):
                        cols = g * 16 + iota16
                        a = tile[k, pl.ds(g * 16, 16)]
                        bt = tile_t[k, pl.ds(g * 16, 16)]
                        tile[k, pl.ds(g * 16, 16)] = jnp.where(
                            cols >= k, a, bt
                        )
                    return c

                lax.fori_loop(0, T, merge, 0)

            i0a = pl.multiple_of(i0, T)
            j0a = pl.multiple_of(j0, T)
            pltpu.async_copy(
                tile, out_hbm.at[pl.ds(i0a, T), pl.ds(j0a, T)], sem_outs[par]
            )

            @pl.when(jnp.logical_not(diag))
            def _():
                pltpu.async_copy(
                    tile_t,
                    out_hbm.at[pl.ds(j0a, T), pl.ds(i0a, T)],
                    sem_outs[par],
                )

    def two_steps(q, carry):
        step(2 * q, 0)

        @pl.when(2 * q + 1 < STEPS)
        def _():
            step(2 * q + 1, 1)

        return carry

    fire(0, 0)
    lax.fori_loop(0, (STEPS + 1) // 2, two_steps, 0)

    for mm in (STEPS - 2, STEPS - 1):
        fi, fj, fvalid = decode(mm)

        @pl.when(fvalid)
        def _():
            wait_tilebytes(mm & 1, tiles[mm & 1])

            @pl.when(fi != fj)
            def _():
                wait_tilebytes(mm & 1, tts[mm & 1])


@jax.jit
def kernel(upper_tri_vector):
    v_pad = jnp.concatenate(
        [upper_tri_vector, jnp.zeros((PAD,), upper_tri_vector.dtype)]
    )
    mesh = plsc.VectorSubcoreMesh(
        core_axis_name="c", subcore_axis_name="s", num_cores=NC
    )
    fn = pl.kernel(
        _body,
        out_type=jax.ShapeDtypeStruct((N, N), jnp.float32),
        mesh=mesh,
        scratch_types=[
            pltpu.VMEM((T * STRIPW,), jnp.float32),
            pltpu.VMEM((T * STRIPW,), jnp.float32),
            pltpu.VMEM((T, T), jnp.float32),
            pltpu.VMEM((T, T), jnp.float32),
            pltpu.VMEM((T, T), jnp.float32),
            pltpu.VMEM((T, T), jnp.float32),
            pltpu.SemaphoreType.DMA,
            pltpu.SemaphoreType.DMA,
            pltpu.SemaphoreType.DMA,
            pltpu.SemaphoreType.DMA,
        ],
        compiler_params=pltpu.CompilerParams(needs_layout_passes=False),
    )
    return fn(v_pad)
